# baseline (speedup 1.0000x reference)
import jax
import jax.numpy as jnp
from jax.experimental import pallas as pl
from jax.experimental.pallas import tpu as pltpu

_N = 512
_K = 102
_EPS = 1e-5


def _make_A(x, a0, a1, a2, eye):
    n = x.shape[0]
    xt = jnp.transpose(x)
    G = jnp.dot(x, xt, preferred_element_type=jnp.float32)
    Gd = G * eye
    sq_col = jnp.sum(Gd, axis=1, keepdims=True)
    sq_row = jnp.sum(Gd, axis=0, keepdims=True)
    d2 = jnp.maximum(sq_col + sq_row - 2.0 * G, 0.0)
    E = jnp.exp(d2 * (-1.0 / 9.0))
    bits = jax.lax.bitcast_convert_type(E, jnp.int32)

    t = jnp.zeros((1, n), jnp.int32)
    kf = float(_K)
    for step in range(15):
        sh = 28 - 2 * step
        t1 = t | (jnp.int32(1) << sh)
        t2 = t | (jnp.int32(2) << sh)
        t3 = t | (jnp.int32(3) << sh)
        c1 = jnp.sum((bits >= t1).astype(jnp.float32), axis=0, keepdims=True)
        c2 = jnp.sum((bits >= t2).astype(jnp.float32), axis=0, keepdims=True)
        c3 = jnp.sum((bits >= t3).astype(jnp.float32), axis=0, keepdims=True)
        t = jnp.where(c1 >= kf, t1, t)
        t = jnp.where(c2 >= kf, t2, t)
        t = jnp.where(c3 >= kf, t3, t)
    kth = t

    mask = (bits >= kth).astype(jnp.float32) * (1.0 - eye)
    mask = mask * jnp.transpose(mask)
    adj = eye + mask * E
    deg_col = jnp.sum(adj, axis=1, keepdims=True) + 1.0
    deg_row = jnp.sum(adj, axis=0, keepdims=True) + 1.0
    An = adj * (1.0 / jnp.sqrt(deg_col)) * (1.0 / jnp.sqrt(deg_row))
    An2 = jnp.dot(An, An, preferred_element_type=jnp.float32)
    return a0 * eye + a1 * An + a2 * An2


def _fused(feat_ref, g1_ref, b1_ref, m1_ref, v1_ref,
           g2_ref, b2_ref, m2_ref, v2_ref,
           w_ref, bias_ref, aifa_ref, out_ref):
    n = _N
    ri = jax.lax.broadcasted_iota(jnp.int32, (n, n), 0)
    ci = jax.lax.broadcasted_iota(jnp.int32, (n, n), 1)
    eye = (ri == ci).astype(jnp.float32)
    a0 = aifa_ref[0]
    a1 = aifa_ref[1]
    a2 = aifa_ref[2]

    feat = feat_ref[...]
    A = _make_A(feat, a0, a1, a2, eye)
    h = jnp.dot(A, feat, preferred_element_type=jnp.float32)
    x = (h - m1_ref[...]) / jnp.sqrt(v1_ref[...] + _EPS) * g1_ref[...] + b1_ref[...]
    x = jnp.maximum(x, 0.0)

    A = _make_A(x, a0, a1, a2, eye)
    support = jnp.dot(x, w_ref[...], preferred_element_type=jnp.float32)
    out = jnp.dot(A, support, preferred_element_type=jnp.float32) + bias_ref[...]
    out = (out - m2_ref[...]) / jnp.sqrt(v2_ref[...] + _EPS) * g2_ref[...] + b2_ref[...]
    out_ref[...] = jnp.maximum(out, 0.0)


def kernel(features, bn1_gamma, bn1_beta, bn1_mean, bn1_var,
           bn2_gamma, bn2_beta, bn2_mean, bn2_var,
           gcn_weight, gcn_bias, aifa1, aifa2, aifa3):
    hid = gcn_weight.shape[1]
    aifa = jax.nn.softmax(jnp.concatenate([aifa1, aifa2, aifa3], axis=0))
    return pl.pallas_call(
        _fused,
        out_shape=jax.ShapeDtypeStruct((_N, hid), jnp.float32),
        in_specs=[pl.BlockSpec(memory_space=pltpu.VMEM)] * 11
        + [pl.BlockSpec(memory_space=pltpu.SMEM)],
        out_specs=pl.BlockSpec(memory_space=pltpu.VMEM),
    )(features, bn1_gamma, bn1_beta, bn1_mean, bn1_var,
      bn2_gamma, bn2_beta, bn2_mean, bn2_var,
      gcn_weight, gcn_bias, aifa)


# speedup vs baseline: 1.1304x; 1.1304x over previous
import jax
import jax.numpy as jnp
from jax.experimental import pallas as pl
from jax.experimental.pallas import tpu as pltpu

_N = 512
_K = 102
_EPS = 1e-5


def _make_A(x, a0, a1, a2, eye):
    n = x.shape[0]
    xt = jnp.transpose(x)
    G = jnp.dot(x, xt, preferred_element_type=jnp.float32)
    Gd = G * eye
    sq_col = jnp.sum(Gd, axis=1, keepdims=True)
    sq_row = jnp.sum(Gd, axis=0, keepdims=True)
    d2 = jnp.maximum(sq_col + sq_row - 2.0 * G, 0.0)
    E = jnp.exp(d2 * (-1.0 / 9.0))
    bits = jax.lax.bitcast_convert_type(E, jnp.int32)

    def colsum(m):
        rows = m.shape[0]
        while rows > 8:
            half = rows // 2
            m = m[:half] + m[half:]
            rows = half
        return jnp.sum(m, axis=0, keepdims=True)

    t = jnp.zeros((1, n), jnp.int32)
    kf = float(_K)
    for step in range(30):
        trial = t | (jnp.int32(1) << (29 - step))
        cnt = colsum((bits >= trial).astype(jnp.float32))
        t = jnp.where(cnt >= kf, trial, t)
    kth = t

    mask = (bits >= kth).astype(jnp.float32) * (1.0 - eye)
    mask = mask * jnp.transpose(mask)
    adj = eye + mask * E
    deg_col = jnp.sum(adj, axis=1, keepdims=True) + 1.0
    deg_row = jnp.sum(adj, axis=0, keepdims=True) + 1.0
    An = adj * (1.0 / jnp.sqrt(deg_col)) * (1.0 / jnp.sqrt(deg_row))
    An2 = jnp.dot(An, An, preferred_element_type=jnp.float32)
    return a0 * eye + a1 * An + a2 * An2


def _fused(feat_ref, g1_ref, b1_ref, m1_ref, v1_ref,
           g2_ref, b2_ref, m2_ref, v2_ref,
           w_ref, bias_ref, aifa_ref, out_ref):
    n = _N
    ri = jax.lax.broadcasted_iota(jnp.int32, (n, n), 0)
    ci = jax.lax.broadcasted_iota(jnp.int32, (n, n), 1)
    eye = (ri == ci).astype(jnp.float32)
    a0 = aifa_ref[0]
    a1 = aifa_ref[1]
    a2 = aifa_ref[2]

    feat = feat_ref[...]
    A = _make_A(feat, a0, a1, a2, eye)
    h = jnp.dot(A, feat, preferred_element_type=jnp.float32)
    x = (h - m1_ref[...]) / jnp.sqrt(v1_ref[...] + _EPS) * g1_ref[...] + b1_ref[...]
    x = jnp.maximum(x, 0.0)

    A = _make_A(x, a0, a1, a2, eye)
    support = jnp.dot(x, w_ref[...], preferred_element_type=jnp.float32)
    out = jnp.dot(A, support, preferred_element_type=jnp.float32) + bias_ref[...]
    out = (out - m2_ref[...]) / jnp.sqrt(v2_ref[...] + _EPS) * g2_ref[...] + b2_ref[...]
    out_ref[...] = jnp.maximum(out, 0.0)


def kernel(features, bn1_gamma, bn1_beta, bn1_mean, bn1_var,
           bn2_gamma, bn2_beta, bn2_mean, bn2_var,
           gcn_weight, gcn_bias, aifa1, aifa2, aifa3):
    hid = gcn_weight.shape[1]
    aifa = jax.nn.softmax(jnp.concatenate([aifa1, aifa2, aifa3], axis=0))
    return pl.pallas_call(
        _fused,
        out_shape=jax.ShapeDtypeStruct((_N, hid), jnp.float32),
        in_specs=[pl.BlockSpec(memory_space=pltpu.VMEM)] * 11
        + [pl.BlockSpec(memory_space=pltpu.SMEM)],
        out_specs=pl.BlockSpec(memory_space=pltpu.VMEM),
    )(features, bn1_gamma, bn1_beta, bn1_mean, bn1_var,
      bn2_gamma, bn2_beta, bn2_mean, bn2_var,
      gcn_weight, gcn_bias, aifa)


# speedup vs baseline: 1.2677x; 1.1214x over previous
import jax
import jax.numpy as jnp
from jax.experimental import pallas as pl
from jax.experimental.pallas import tpu as pltpu

_N = 512
_K = 102
_EPS = 1e-5


def _make_A(x, a0, a1, a2, eye):
    n = x.shape[0]
    xt = jnp.transpose(x)
    G = jnp.dot(x, xt, preferred_element_type=jnp.float32)
    Gd = G * eye
    sq_col = jnp.sum(Gd, axis=1, keepdims=True)
    sq_row = jnp.sum(Gd, axis=0, keepdims=True)
    d2 = jnp.maximum(sq_col + sq_row - 2.0 * G, 0.0)
    E = jnp.exp(d2 * (-1.0 / 9.0))
    bits = jax.lax.bitcast_convert_type(E, jnp.int32)

    hi = (bits >> 16).astype(jnp.int16)
    lob = ((bits & 0xFFFF) - 32768).astype(jnp.int16)

    def csum16(m):
        rows = m.shape[0]
        while rows > 1:
            half = rows // 2
            m = m[:half] + m[half:]
            rows = half
        return m

    k16 = jnp.int16(_K)
    t = jnp.zeros((1, n), jnp.int16)
    for step in range(14):
        trial = t | jnp.int16(1 << (13 - step))
        cnt = csum16((hi >= trial).astype(jnp.int16))
        t = jnp.where(cnt >= k16, trial, t)

    g = csum16((hi > t).astype(jnp.int16))
    band = (hi == t).astype(jnp.int16)
    need = (k16 - g).astype(jnp.int32)
    zero16 = jnp.int16(0)
    tu = jnp.zeros((1, n), jnp.int32)
    for step in range(16):
        trial_u = tu | (1 << (15 - step))
        trial_b = (trial_u - 32768).astype(jnp.int16)
        cnt = csum16(jnp.where(lob >= trial_b, band, zero16))
        tu = jnp.where(cnt.astype(jnp.int32) >= need, trial_u, tu)
    kth = (t.astype(jnp.int32) << 16) | tu

    mask = (bits >= kth).astype(jnp.float32) * (1.0 - eye)
    mask = mask * jnp.transpose(mask)
    adj = eye + mask * E
    deg_col = jnp.sum(adj, axis=1, keepdims=True) + 1.0
    deg_row = jnp.sum(adj, axis=0, keepdims=True) + 1.0
    An = adj * (1.0 / jnp.sqrt(deg_col)) * (1.0 / jnp.sqrt(deg_row))
    An2 = jnp.dot(An, An, preferred_element_type=jnp.float32)
    return a0 * eye + a1 * An + a2 * An2


def _fused(feat_ref, g1_ref, b1_ref, m1_ref, v1_ref,
           g2_ref, b2_ref, m2_ref, v2_ref,
           w_ref, bias_ref, aifa_ref, out_ref):
    n = _N
    ri = jax.lax.broadcasted_iota(jnp.int32, (n, n), 0)
    ci = jax.lax.broadcasted_iota(jnp.int32, (n, n), 1)
    eye = (ri == ci).astype(jnp.float32)
    a0 = aifa_ref[0]
    a1 = aifa_ref[1]
    a2 = aifa_ref[2]

    feat = feat_ref[...]
    A = _make_A(feat, a0, a1, a2, eye)
    h = jnp.dot(A, feat, preferred_element_type=jnp.float32)
    x = (h - m1_ref[...]) / jnp.sqrt(v1_ref[...] + _EPS) * g1_ref[...] + b1_ref[...]
    x = jnp.maximum(x, 0.0)

    A = _make_A(x, a0, a1, a2, eye)
    support = jnp.dot(x, w_ref[...], preferred_element_type=jnp.float32)
    out = jnp.dot(A, support, preferred_element_type=jnp.float32) + bias_ref[...]
    out = (out - m2_ref[...]) / jnp.sqrt(v2_ref[...] + _EPS) * g2_ref[...] + b2_ref[...]
    out_ref[...] = jnp.maximum(out, 0.0)


def kernel(features, bn1_gamma, bn1_beta, bn1_mean, bn1_var,
           bn2_gamma, bn2_beta, bn2_mean, bn2_var,
           gcn_weight, gcn_bias, aifa1, aifa2, aifa3):
    hid = gcn_weight.shape[1]
    aifa = jax.nn.softmax(jnp.concatenate([aifa1, aifa2, aifa3], axis=0))
    return pl.pallas_call(
        _fused,
        out_shape=jax.ShapeDtypeStruct((_N, hid), jnp.float32),
        in_specs=[pl.BlockSpec(memory_space=pltpu.VMEM)] * 11
        + [pl.BlockSpec(memory_space=pltpu.SMEM)],
        out_specs=pl.BlockSpec(memory_space=pltpu.VMEM),
    )(features, bn1_gamma, bn1_beta, bn1_mean, bn1_var,
      bn2_gamma, bn2_beta, bn2_mean, bn2_var,
      gcn_weight, gcn_bias, aifa)


# speedup vs baseline: 1.3975x; 1.1024x over previous
import jax
import jax.numpy as jnp
from jax.experimental import pallas as pl
from jax.experimental.pallas import tpu as pltpu

_N = 512
_K = 102
_EPS = 1e-5


def _make_A(x, a0, a1, a2, eye):
    n = x.shape[0]
    xt = jnp.transpose(x)
    G = jnp.dot(x, xt, preferred_element_type=jnp.float32)
    Gd = G * eye
    sq_col = jnp.sum(Gd, axis=1, keepdims=True)
    sq_row = jnp.sum(Gd, axis=0, keepdims=True)
    d2 = jnp.maximum(sq_col + sq_row - 2.0 * G, 0.0)
    E = jnp.exp(d2 * (-1.0 / 9.0))
    bits = jax.lax.bitcast_convert_type(E, jnp.int32)

    hi = (bits >> 16).astype(jnp.int16)
    lob = ((bits & 0xFFFF) - 32768).astype(jnp.int16)

    def csum16(m):
        rows = m.shape[0]
        while rows > 1:
            half = rows // 2
            m = m[:half] + m[half:]
            rows = half
        return m

    k16 = jnp.int16(_K)
    t = jnp.zeros((1, n), jnp.int16)
    for step in range(14):
        trial = t | jnp.int16(1 << (13 - step))
        cnt = csum16((hi >= trial).astype(jnp.int16))
        t = jnp.where(cnt >= k16, trial, t)

    g = csum16((hi > t).astype(jnp.int16))
    band = (hi == t).astype(jnp.int16)
    need = (k16 - g).astype(jnp.int32)
    zero16 = jnp.int16(0)
    tu = jnp.zeros((1, n), jnp.int32)
    for step in range(16):
        trial_u = tu | (1 << (15 - step))
        trial_b = (trial_u - 32768).astype(jnp.int16)
        cnt = csum16(jnp.where(lob >= trial_b, band, zero16))
        tu = jnp.where(cnt.astype(jnp.int32) >= need, trial_u, tu)
    kth = (t.astype(jnp.int32) << 16) | tu

    mask = (bits >= kth).astype(jnp.float32) * (1.0 - eye)
    mask = mask * jnp.transpose(mask)
    adj = eye + mask * E
    deg_col = jnp.sum(adj, axis=1, keepdims=True) + 1.0
    deg_row = jnp.sum(adj, axis=0, keepdims=True) + 1.0
    An = adj * (1.0 / jnp.sqrt(deg_col)) * (1.0 / jnp.sqrt(deg_row))
    An2 = jnp.dot(An, An, preferred_element_type=jnp.float32)
    return a0 * eye + a1 * An + a2 * An2


def _fused(feat_ref, g1_ref, b1_ref, m1_ref, v1_ref,
           g2_ref, b2_ref, m2_ref, v2_ref,
           w_ref, bias_ref, aifa1_ref, aifa2_ref, aifa3_ref, out_ref):
    n = _N
    ri = jax.lax.broadcasted_iota(jnp.int32, (n, n), 0)
    ci = jax.lax.broadcasted_iota(jnp.int32, (n, n), 1)
    eye = (ri == ci).astype(jnp.float32)
    s1 = aifa1_ref[0]
    s2 = aifa2_ref[0]
    s3 = aifa3_ref[0]
    sm = jnp.maximum(jnp.maximum(s1, s2), s3)
    e1 = jnp.exp(s1 - sm)
    e2 = jnp.exp(s2 - sm)
    e3 = jnp.exp(s3 - sm)
    es = e1 + e2 + e3
    a0 = e1 / es
    a1 = e2 / es
    a2 = e3 / es

    feat = feat_ref[...]
    A = _make_A(feat, a0, a1, a2, eye)
    h = jnp.dot(A, feat, preferred_element_type=jnp.float32)
    x = (h - m1_ref[...]) / jnp.sqrt(v1_ref[...] + _EPS) * g1_ref[...] + b1_ref[...]
    x = jnp.maximum(x, 0.0)

    A = _make_A(x, a0, a1, a2, eye)
    support = jnp.dot(x, w_ref[...], preferred_element_type=jnp.float32)
    out = jnp.dot(A, support, preferred_element_type=jnp.float32) + bias_ref[...]
    out = (out - m2_ref[...]) / jnp.sqrt(v2_ref[...] + _EPS) * g2_ref[...] + b2_ref[...]
    out_ref[...] = jnp.maximum(out, 0.0)


def kernel(features, bn1_gamma, bn1_beta, bn1_mean, bn1_var,
           bn2_gamma, bn2_beta, bn2_mean, bn2_var,
           gcn_weight, gcn_bias, aifa1, aifa2, aifa3):
    hid = gcn_weight.shape[1]
    return pl.pallas_call(
        _fused,
        out_shape=jax.ShapeDtypeStruct((_N, hid), jnp.float32),
        in_specs=[pl.BlockSpec(memory_space=pltpu.VMEM)] * 11
        + [pl.BlockSpec(memory_space=pltpu.SMEM)] * 3,
        out_specs=pl.BlockSpec(memory_space=pltpu.VMEM),
    )(features, bn1_gamma, bn1_beta, bn1_mean, bn1_var,
      bn2_gamma, bn2_beta, bn2_mean, bn2_var,
      gcn_weight, gcn_bias, aifa1, aifa2, aifa3)


# speedup vs baseline: 1.4619x; 1.0461x over previous
import jax
import jax.numpy as jnp
from jax.experimental import pallas as pl
from jax.experimental.pallas import tpu as pltpu

_N = 512
_K = 102
_EPS = 1e-5


def _make_An(x, eye, neye):
    n = x.shape[0]
    xt = jnp.transpose(x)
    G = jnp.dot(x, xt, preferred_element_type=jnp.float32)
    Gd = G * eye
    sq_col = jnp.sum(Gd, axis=1, keepdims=True)
    sq_row = jnp.sum(Gd, axis=0, keepdims=True)
    d2 = jnp.maximum(sq_col + sq_row - 2.0 * G, 0.0)
    E = jnp.exp(d2 * (-1.0 / 9.0))
    bits = jax.lax.bitcast_convert_type(E, jnp.int32)

    hi = (bits >> 16).astype(jnp.int16)
    lob = ((bits & 0xFFFF) - 32768).astype(jnp.int16)

    nch = 8
    crows = n // nch

    def count_cols(x, pred):
        chunks = [x[i * crows:(i + 1) * crows] for i in range(nch)]
        acc = pred(chunks[0])
        for c in chunks[1:]:
            acc = acc + pred(c)
        while acc.shape[0] > 1:
            h = acc.shape[0] // 2
            acc = acc[:h] + acc[h:]
        return acc

    one16 = jnp.int16(1)
    zero16 = jnp.int16(0)

    k16 = jnp.int16(_K)
    t = jnp.zeros((1, n), jnp.int16)
    for step in range(14):
        trial = t | jnp.int16(1 << (13 - step))
        cnt = count_cols(hi, lambda c: jnp.where(c >= trial, one16, zero16))
        t = jnp.where(cnt >= k16, trial, t)

    g = count_cols(hi, lambda c: jnp.where(c > t, one16, zero16))
    lobm = jnp.where(hi == t, lob, jnp.int16(-32768))
    need = (k16 - g).astype(jnp.int32)
    tu = jnp.zeros((1, n), jnp.int32)
    for step in range(16):
        trial_u = tu | (1 << (15 - step))
        trial_b = (trial_u - 32768).astype(jnp.int16)
        cnt = count_cols(lobm, lambda c: jnp.where(c >= trial_b, one16, zero16))
        tu = jnp.where(cnt.astype(jnp.int32) >= need, trial_u, tu)
    kth = (t.astype(jnp.int32) << 16) | tu

    mask = (bits >= kth).astype(jnp.float32) * neye
    mask = mask * jnp.transpose(mask)
    adj = eye + mask * E
    deg_col = jnp.sum(adj, axis=1, keepdims=True) + 1.0
    deg_row = jnp.sum(adj, axis=0, keepdims=True) + 1.0
    return adj * (1.0 / jnp.sqrt(deg_col)) * (1.0 / jnp.sqrt(deg_row))


def _apply_A(An, X, a0, a1, a2):
    y1 = jnp.dot(An, X, preferred_element_type=jnp.float32)
    y2 = jnp.dot(An, y1, preferred_element_type=jnp.float32)
    return a0 * X + a1 * y1 + a2 * y2


def _fused(feat_ref, g1_ref, b1_ref, m1_ref, v1_ref,
           g2_ref, b2_ref, m2_ref, v2_ref,
           w_ref, bias_ref, aifa1_ref, aifa2_ref, aifa3_ref, out_ref):
    n = _N
    ri = jax.lax.broadcasted_iota(jnp.int32, (n, n), 0)
    ci = jax.lax.broadcasted_iota(jnp.int32, (n, n), 1)
    eye = (ri == ci).astype(jnp.float32)
    neye = 1.0 - eye
    s1 = aifa1_ref[0]
    s2 = aifa2_ref[0]
    s3 = aifa3_ref[0]
    sm = jnp.maximum(jnp.maximum(s1, s2), s3)
    e1 = jnp.exp(s1 - sm)
    e2 = jnp.exp(s2 - sm)
    e3 = jnp.exp(s3 - sm)
    es = e1 + e2 + e3
    a0 = e1 / es
    a1 = e2 / es
    a2 = e3 / es

    feat = feat_ref[...]
    An = _make_An(feat, eye, neye)
    h = _apply_A(An, feat, a0, a1, a2)
    x = (h - m1_ref[...]) / jnp.sqrt(v1_ref[...] + _EPS) * g1_ref[...] + b1_ref[...]
    x = jnp.maximum(x, 0.0)

    An = _make_An(x, eye, neye)
    An2 = jnp.dot(An, An, preferred_element_type=jnp.float32)
    A2 = a0 * eye + a1 * An + a2 * An2
    support = jnp.dot(x, w_ref[...], preferred_element_type=jnp.float32)
    out = jnp.dot(A2, support, preferred_element_type=jnp.float32) + bias_ref[...]
    out = (out - m2_ref[...]) / jnp.sqrt(v2_ref[...] + _EPS) * g2_ref[...] + b2_ref[...]
    out_ref[...] = jnp.maximum(out, 0.0)


def kernel(features, bn1_gamma, bn1_beta, bn1_mean, bn1_var,
           bn2_gamma, bn2_beta, bn2_mean, bn2_var,
           gcn_weight, gcn_bias, aifa1, aifa2, aifa3):
    hid = gcn_weight.shape[1]
    return pl.pallas_call(
        _fused,
        out_shape=jax.ShapeDtypeStruct((_N, hid), jnp.float32),
        in_specs=[pl.BlockSpec(memory_space=pltpu.VMEM)] * 11
        + [pl.BlockSpec(memory_space=pltpu.SMEM)] * 3,
        out_specs=pl.BlockSpec(memory_space=pltpu.VMEM),
    )(features, bn1_gamma, bn1_beta, bn1_mean, bn1_var,
      bn2_gamma, bn2_beta, bn2_mean, bn2_var,
      gcn_weight, gcn_bias, aifa1, aifa2, aifa3)
